# baseline (speedup 1.0000x reference)
import functools

import jax
import jax.numpy as jnp
from jax import lax
from jax.experimental import pallas as pl
from jax.experimental.pallas import tpu as pltpu
from jax.experimental.pallas import tpu_sc as plsc

B, N, L, D = 8, 4, 2048, 768
G = 320
NUM_TILES = 32
TOK = B * L
TPT = TOK // NUM_TILES
BL = 1024
NJ = L // BL


def _sc_gather(vq, wflat):
    mesh = plsc.VectorSubcoreMesh(core_axis_name="c", subcore_axis_name="s")

    @functools.partial(
        pl.kernel,
        out_type=jax.ShapeDtypeStruct((B, NJ, 1, BL), jnp.float32),
        mesh=mesh,
        scratch_types=[
            pltpu.VMEM((TPT, 2), jnp.int32),
            pltpu.VMEM((TPT,), jnp.int32),
            pltpu.VMEM((TPT,), jnp.float32),
            pltpu.SemaphoreType.DMA,
            pltpu.SemaphoreType.DMA,
        ],
        compiler_params=pltpu.CompilerParams(needs_layout_passes=False),
    )
    def k(vq_hbm, wt_hbm, w_hbm, idx2_v, idx_v, w_v, sem0, sem1):
        sid = lax.axis_index("s")
        wid = sid * 2 + lax.axis_index("c")
        spb = L // TPT
        b = wid // spb
        q = wid % spb
        jblk = q // (BL // TPT)
        off = (q % (BL // TPT)) * TPT

        cp1 = pltpu.make_async_copy(
            vq_hbm.at[b, pl.ds(q * TPT, TPT)], idx2_v, sem1)
        cp1.start()
        cp1.wait()
        iot = lax.iota(jnp.int32, 16)
        zero16 = jnp.zeros((16,), jnp.int32)
        one16 = jnp.ones((16,), jnp.int32)
        for j in range(TPT // 16):
            rows = j * 16 + iot
            i0 = plsc.load_gather(idx2_v, [rows, zero16])
            i1 = plsc.load_gather(idx2_v, [rows, one16])
            idx_v[pl.ds(j * 16, 16)] = i0 * G + i1
        for t in range(TPT // 128):
            pltpu.make_async_copy(
                wt_hbm.at[idx_v.at[pl.ds(t * 128, 128)]],
                w_v.at[pl.ds(t * 128, 128)], sem0).start()
        for t in range(TPT // 128):
            pltpu.make_async_copy(
                wt_hbm.at[idx_v.at[pl.ds(t * 128, 128)]],
                w_v.at[pl.ds(t * 128, 128)], sem0).wait()
        pltpu.sync_copy(w_v, w_hbm.at[b, jblk, 0, pl.ds(off, TPT)])

    return k(vq, wflat)


def _tc_reduce(x_full, w4, lens):
    NBUF = 5
    SUB = 64
    NS = BL // SUB

    def body(lens_ref, w_ref, x_ref, o_ref, *scratch):
        bufs = scratch[:NBUF]
        sems = scratch[NBUF:]
        o_ref[...] = jnp.zeros_like(o_ref)
        nb = [(lens_ref[b] + BL - 1) // BL for b in range(B)]
        nbf = [lens_ref[b] // BL for b in range(B)]

        def transfers(s):
            b, j = divmod(s, NJ)
            m = s % NBUF
            full = pltpu.make_async_copy(
                x_ref.at[b, N - 1, pl.ds(j * BL, BL), :], bufs[m], sems[m])
            rem = lens_ref[b] - j * BL
            subs = [
                (k * SUB < rem,
                 pltpu.make_async_copy(
                     x_ref.at[b, N - 1, pl.ds(j * BL + k * SUB, SUB), :],
                     bufs[m].at[pl.ds(k * SUB, SUB), :], sems[m]))
                for k in range(NS)
            ]
            return b, j, full, subs

        def start_slot(s):
            b, j, full, subs = transfers(s)

            @pl.when(j < nbf[b])
            def _():
                full.start()

            @pl.when((j == nbf[b]) & (j < nb[b]))
            def _():
                for ok, cp in subs:
                    @pl.when(ok)
                    def _(cp=cp):
                        cp.start()

        def wait_slot(s):
            b, j, full, subs = transfers(s)

            @pl.when(j < nbf[b])
            def _():
                full.wait()

            @pl.when((j == nbf[b]) & (j < nb[b]))
            def _():
                for ok, cp in subs:
                    @pl.when(ok)
                    def _(cp=cp):
                        cp.wait()

        for s in range(NBUF - 1):
            start_slot(s)

        pos = lax.broadcasted_iota(jnp.int32, (1, BL), 1)
        for s in range(B * NJ):
            b, j = divmod(s, NJ)
            wait_slot(s)
            if s + NBUF - 1 < B * NJ:
                start_slot(s + NBUF - 1)

            @pl.when(j < nb[b])
            def _(b=b, j=j, m=s % NBUF):
                wv = jnp.where(
                    j * BL + pos < lens_ref[b], w_ref[b, j],
                    jnp.zeros((1, BL), jnp.float32))
                o_ref[b:b + 1, :] += lax.dot_general(
                    wv, bufs[m][...], (((1,), (0,)), ((), ())),
                    preferred_element_type=jnp.float32)

    grid_spec = pltpu.PrefetchScalarGridSpec(
        num_scalar_prefetch=1,
        grid=(1,),
        in_specs=[
            pl.BlockSpec((B, NJ, 1, BL), lambda i, lens: (0, 0, 0, 0)),
            pl.BlockSpec(memory_space=pl.ANY),
        ],
        out_specs=pl.BlockSpec((B, D), lambda i, lens: (0, 0)),
        scratch_shapes=(
            [pltpu.VMEM((BL, D), jnp.float32) for _ in range(NBUF)]
            + [pltpu.SemaphoreType.DMA for _ in range(NBUF)]
        ),
    )
    return pl.pallas_call(
        body,
        grid_spec=grid_spec,
        out_shape=jax.ShapeDtypeStruct((B, D), jnp.float32),
    )(lens, w4, x_full)


def kernel(input_feature, input_lengths, vq_indices, weight):
    lens = input_lengths.astype(jnp.int32)
    w4 = _sc_gather(vq_indices, weight.reshape(-1))
    return _tc_reduce(input_feature, w4, lens)


# speedup vs baseline: 1.2297x; 1.2297x over previous
import functools

import jax
import jax.numpy as jnp
from jax import lax
from jax.experimental import pallas as pl
from jax.experimental.pallas import tpu as pltpu
from jax.experimental.pallas import tpu_sc as plsc

B, N, L, D = 8, 4, 2048, 768
G = 320
NUM_TILES = 32
TOK = B * L
TPT = TOK // NUM_TILES
BL = 1024
NJ = L // BL


def _sc_gather(fidx, wflat):
    mesh = plsc.VectorSubcoreMesh(core_axis_name="c", subcore_axis_name="s")

    @functools.partial(
        pl.kernel,
        out_type=jax.ShapeDtypeStruct((B, NJ, 1, BL), jnp.float32),
        mesh=mesh,
        scratch_types=[
            pltpu.VMEM((TPT,), jnp.int32),
            pltpu.VMEM((TPT,), jnp.float32),
            pltpu.SemaphoreType.DMA,
            pltpu.SemaphoreType.DMA,
        ],
        compiler_params=pltpu.CompilerParams(needs_layout_passes=False),
    )
    def k(fidx_hbm, wt_hbm, w_hbm, idx_v, w_v, sem0, sem1):
        sid = lax.axis_index("s")
        wid = sid * 2 + lax.axis_index("c")
        spb = L // TPT
        b = wid // spb
        q = wid % spb
        jblk = q // (BL // TPT)
        off = (q % (BL // TPT)) * TPT

        cp1 = pltpu.make_async_copy(
            fidx_hbm.at[pl.ds(wid * TPT, TPT)], idx_v, sem1)
        cp1.start()
        cp1.wait()
        for t in range(TPT // 128):
            pltpu.make_async_copy(
                wt_hbm.at[idx_v.at[pl.ds(t * 128, 128)]],
                w_v.at[pl.ds(t * 128, 128)], sem0).start()
        for t in range(TPT // 128):
            pltpu.make_async_copy(
                wt_hbm.at[idx_v.at[pl.ds(t * 128, 128)]],
                w_v.at[pl.ds(t * 128, 128)], sem0).wait()
        pltpu.sync_copy(w_v, w_hbm.at[b, jblk, 0, pl.ds(off, TPT)])

    return k(fidx, wflat)


def _tc_reduce(x_full, w4, lens):
    NBUF = 5
    SUB = 64
    NS = BL // SUB

    def body(lens_ref, w_ref, x_ref, o_ref, *scratch):
        bufs = scratch[:NBUF]
        sems = scratch[NBUF:]
        o_ref[...] = jnp.zeros_like(o_ref)
        nb = [(lens_ref[b] + BL - 1) // BL for b in range(B)]
        nbf = [lens_ref[b] // BL for b in range(B)]

        def transfers(s):
            b, j = divmod(s, NJ)
            m = s % NBUF
            full = pltpu.make_async_copy(
                x_ref.at[b, N - 1, pl.ds(j * BL, BL), :], bufs[m], sems[m])
            rem = lens_ref[b] - j * BL
            subs = [
                (k * SUB < rem,
                 pltpu.make_async_copy(
                     x_ref.at[b, N - 1, pl.ds(j * BL + k * SUB, SUB), :],
                     bufs[m].at[pl.ds(k * SUB, SUB), :], sems[m]))
                for k in range(NS)
            ]
            return b, j, full, subs

        def start_slot(s):
            b, j, full, subs = transfers(s)

            @pl.when(j < nbf[b])
            def _():
                full.start()

            @pl.when((j == nbf[b]) & (j < nb[b]))
            def _():
                for ok, cp in subs:
                    @pl.when(ok)
                    def _(cp=cp):
                        cp.start()

        def wait_slot(s):
            b, j, full, subs = transfers(s)

            @pl.when(j < nbf[b])
            def _():
                full.wait()

            @pl.when((j == nbf[b]) & (j < nb[b]))
            def _():
                for ok, cp in subs:
                    @pl.when(ok)
                    def _(cp=cp):
                        cp.wait()

        for s in range(NBUF - 1):
            start_slot(s)

        pos = lax.broadcasted_iota(jnp.int32, (1, BL), 1)
        for s in range(B * NJ):
            b, j = divmod(s, NJ)
            wait_slot(s)
            if s + NBUF - 1 < B * NJ:
                start_slot(s + NBUF - 1)

            @pl.when(j < nb[b])
            def _(b=b, j=j, m=s % NBUF):
                wv = jnp.where(
                    j * BL + pos < lens_ref[b], w_ref[b, j],
                    jnp.zeros((1, BL), jnp.float32))
                o_ref[b:b + 1, :] += lax.dot_general(
                    wv, bufs[m][...], (((1,), (0,)), ((), ())),
                    preferred_element_type=jnp.float32)

    grid_spec = pltpu.PrefetchScalarGridSpec(
        num_scalar_prefetch=1,
        grid=(1,),
        in_specs=[
            pl.BlockSpec((B, NJ, 1, BL), lambda i, lens: (0, 0, 0, 0)),
            pl.BlockSpec(memory_space=pl.ANY),
        ],
        out_specs=pl.BlockSpec((B, D), lambda i, lens: (0, 0)),
        scratch_shapes=(
            [pltpu.VMEM((BL, D), jnp.float32) for _ in range(NBUF)]
            + [pltpu.SemaphoreType.DMA for _ in range(NBUF)]
        ),
    )
    return pl.pallas_call(
        body,
        grid_spec=grid_spec,
        out_shape=jax.ShapeDtypeStruct((B, D), jnp.float32),
    )(lens, w4, x_full)


def kernel(input_feature, input_lengths, vq_indices, weight):
    lens = input_lengths.astype(jnp.int32)
    fidx = (vq_indices[..., 0] * G + vq_indices[..., 1]).reshape(-1)
    w4 = _sc_gather(fidx, weight.reshape(-1))
    return _tc_reduce(input_feature, w4, lens)
